# Initial kernel scaffold; baseline (speedup 1.0000x reference)
import jax
import jax.numpy as jnp
from jax.experimental import pallas as pl


def kernel(mu, log_var, weight, epoch):
    raise NotImplementedError("write your pallas kernel here")


# speedup vs baseline: 3.4603x; 3.4603x over previous
import functools

import jax
import jax.numpy as jnp
from jax import lax
from jax.experimental import pallas as pl
from jax.experimental.pallas import tpu as pltpu
from jax.experimental.pallas import tpu_sc as plsc

NC = 2
NS = 16
L = 16
NW = NC * NS

B, K, D = 2048, 16, 2048
TOK = B // NW
NGROUP = TOK // L
CH = 16
NCHUNK = TOK // CH

_f32 = jnp.float32
_i32 = jnp.int32


def _sc_body(mu_hbm, lv_hbm, w_hbm, mask_hbm, rand_hbm, eps_hbm,
             sample_out, idx_out, cmu_out, clv_out,
             wbuf, mbuf, rbuf, cbuf, ibuf, mu_b, lv_b, ep_b,
             sem1, sem2, sem3):
    wid = lax.axis_index("s") * NC + lax.axis_index("c")
    base = wid * TOK

    pltpu.sync_copy(w_hbm.at[wid], wbuf)
    pltpu.sync_copy(mask_hbm.at[wid], mbuf)
    pltpu.sync_copy(rand_hbm.at[wid], rbuf)

    for g in range(NGROUP):
        sl = pl.ds(g * L, L)
        m = wbuf[0, sl]
        am = jnp.zeros((L,), _i32)
        for k in range(1, K):
            vk = wbuf[k, sl]
            gt = vk > m
            am = jnp.where(gt, k, am)
            m = jnp.where(gt, vk, m)
        chosen = jnp.where(mbuf[sl] != 0, rbuf[sl], am)
        tok = base + g * L + lax.iota(_i32, L)
        cbuf[sl] = chosen
        ibuf[g, :] = tok * K + chosen

    pltpu.sync_copy(cbuf, idx_out.at[wid])

    for c in range(NCHUNK):
        t0 = base + c * CH
        cp1 = pltpu.async_copy(mu_hbm.at[ibuf.at[c]], mu_b, sem1)
        cp2 = pltpu.async_copy(lv_hbm.at[ibuf.at[c]], lv_b, sem2)
        cp3 = pltpu.async_copy(eps_hbm.at[pl.ds(t0, CH)], ep_b, sem3)
        cp1.wait()
        cp2.wait()
        cp3.wait()
        pltpu.sync_copy(mu_b, cmu_out.at[pl.ds(t0, CH)])
        pltpu.sync_copy(lv_b, clv_out.at[pl.ds(t0, CH)])
        for r in range(CH):
            def cbody(j, _, r=r):
                s2 = pl.ds(j * L, L)
                ep_b[r, s2] = mu_b[r, s2] + jnp.exp(lv_b[r, s2] * 0.5) * ep_b[r, s2]
                return 0
            lax.fori_loop(0, D // L, cbody, 0)
        pltpu.sync_copy(ep_b, sample_out.at[pl.ds(t0, CH)])


@jax.jit
def _sc_call(mu_flat, lv_flat, w_arr, mask2, rand2, eps):
    mesh = plsc.VectorSubcoreMesh(core_axis_name="c", subcore_axis_name="s")
    fn = functools.partial(
        pl.kernel,
        mesh=mesh,
        out_type=(
            jax.ShapeDtypeStruct((B, D), _f32),
            jax.ShapeDtypeStruct((NW, TOK), _i32),
            jax.ShapeDtypeStruct((B, D), _f32),
            jax.ShapeDtypeStruct((B, D), _f32),
        ),
        scratch_types=[
            pltpu.VMEM((K, TOK), _f32),
            pltpu.VMEM((TOK,), _i32),
            pltpu.VMEM((TOK,), _i32),
            pltpu.VMEM((TOK,), _i32),
            pltpu.VMEM((NCHUNK, CH), _i32),
            pltpu.VMEM((CH, D), _f32),
            pltpu.VMEM((CH, D), _f32),
            pltpu.VMEM((CH, D), _f32),
            pltpu.SemaphoreType.DMA,
            pltpu.SemaphoreType.DMA,
            pltpu.SemaphoreType.DMA,
        ],
    )(_sc_body)
    return fn(mu_flat, lv_flat, w_arr, mask2, rand2, eps)


def kernel(mu, log_var, weight, epoch):
    epsilon = 0.9
    rkey = jax.random.key(42)
    km, kr, ke = jax.random.split(rkey, 3)
    mask = jax.random.uniform(km, (B,), dtype=_f32) < epsilon
    rand_idx = jax.random.randint(kr, (B,), 0, K)
    eps = jax.random.normal(ke, (B, D), dtype=_f32)

    mu_flat = mu.reshape(B * K, D)
    lv_flat = log_var.reshape(B * K, D)
    w_arr = jnp.transpose(weight).reshape(K, NW, TOK).transpose(1, 0, 2)
    mask2 = mask.astype(_i32).reshape(NW, TOK)
    rand2 = rand_idx.reshape(NW, TOK)

    sample, idxs, cmu, clv = _sc_call(mu_flat, lv_flat, w_arr, mask2, rand2, eps)
    return sample, idxs.reshape(B), cmu, clv
